# baseline (speedup 1.0000x reference)
import functools

import jax
import jax.numpy as jnp
from jax.experimental import pallas as pl

K_NN = 32
ROW_BLOCK = 256
DROP_BITS = 14


def _mlp_body(pr_ref, w1_ref, b1_ref, w2_ref, b2_ref, w3_ref, b3_ref, f3_ref):
    p = pr_ref[0]
    f = jnp.maximum(jnp.dot(p, w1_ref[...], preferred_element_type=jnp.float32)
                    + b1_ref[...], 0.0)
    f = jnp.maximum(jnp.dot(f, w2_ref[...], preferred_element_type=jnp.float32)
                    + b2_ref[...], 0.0)
    f = jnp.maximum(jnp.dot(f, w3_ref[...], preferred_element_type=jnp.float32)
                    + b3_ref[...], 0.0)
    f3_ref[0] = f.astype(jnp.bfloat16)


def _knn_pool_body(pr_ref, pt_ref, f3_ref, out_ref):
    q = pr_ref[0]
    pt = pt_ref[0]

    g = jnp.dot(q, pt, preferred_element_type=jnp.float32,
                precision=jax.lax.Precision.HIGHEST)
    qn = jnp.sum(q * q, axis=1, keepdims=True)
    pn = jnp.sum(pt * pt, axis=0, keepdims=True)
    d2 = (qn + pn) - (g + g)

    bits = jax.lax.bitcast_convert_type(d2, jnp.int32)
    rb = bits.shape[0]
    n = bits.shape[1]
    ones = jnp.ones((n, 1), jnp.float32)

    hi16 = jnp.right_shift(bits, 16).astype(jnp.int16)

    def body16(i, carry):
        t, m = carry
        cand = t | jnp.left_shift(jnp.int32(1), 14 - i)
        x = (hi16 < cand.astype(jnp.int16)).astype(jnp.int16)
        while x.shape[1] > 128:
            h = x.shape[1] // 2
            x = x[:, :h] + x[:, h:]
        cnt = jnp.sum(x.astype(jnp.int32), axis=1, keepdims=True)
        take = cnt < K_NN
        return jnp.where(take, cand, t), jnp.where(take, cnt, m)

    t16, m16 = jax.lax.fori_loop(
        0, 15, body16,
        (jnp.zeros((rb, 1), jnp.int32), jnp.zeros((rb, 1), jnp.int32)))

    def body(i, carry):
        t, m = carry
        cand = t | jnp.left_shift(jnp.int32(1), 15 - i)
        cnt = jnp.sum((bits < cand).astype(jnp.int32), axis=1, keepdims=True)
        take = cnt < K_NN
        return jnp.where(take, cand, t), jnp.where(take, cnt, m)

    t0 = jnp.left_shift(t16, 16)
    t, m = jax.lax.fori_loop(0, 16 - DROP_BITS, body, (t0, m16))

    ltf = jnp.where(bits < t, 1.0, 0.0)
    lt2f = jnp.where(bits < t + jnp.int32(1 << DROP_BITS), 1.0, 0.0)
    eqf = lt2f - ltf
    e = jnp.sum(eqf, axis=1, keepdims=True)
    tie_w = (float(K_NN) - m.astype(jnp.float32)) / e
    w = (ltf + eqf * tie_w).astype(jnp.bfloat16)

    out_ref[0] = jnp.dot(w, f3_ref[0], preferred_element_type=jnp.float32) \
        * (1.0 / K_NN)


def kernel(points, W1, b1, W2, b2, W3, b3):
    B, N, C = points.shape
    pr = jnp.pad(points, ((0, 0), (0, 0), (0, 8 - C)))
    pt = jnp.swapaxes(pr, 1, 2)
    w1p = jnp.pad(W1, ((0, 8 - C), (0, 0)))

    f3 = pl.pallas_call(
        _mlp_body,
        grid=(B,),
        in_specs=[
            pl.BlockSpec((1, N, 8), lambda b: (b, 0, 0)),
            pl.BlockSpec((8, 64), lambda b: (0, 0)),
            pl.BlockSpec((1, 64), lambda b: (0, 0)),
            pl.BlockSpec((64, 64), lambda b: (0, 0)),
            pl.BlockSpec((1, 64), lambda b: (0, 0)),
            pl.BlockSpec((64, 128), lambda b: (0, 0)),
            pl.BlockSpec((1, 128), lambda b: (0, 0)),
        ],
        out_specs=pl.BlockSpec((1, N, 128), lambda b: (b, 0, 0)),
        out_shape=jax.ShapeDtypeStruct((B, N, 128), jnp.bfloat16),
    )(pr, w1p, b1[None], W2, b2[None], W3, b3[None])

    n_rb = N // ROW_BLOCK
    out = pl.pallas_call(
        _knn_pool_body,
        grid=(B, n_rb),
        in_specs=[
            pl.BlockSpec((1, ROW_BLOCK, 8), lambda b, r: (b, r, 0)),
            pl.BlockSpec((1, 8, N), lambda b, r: (b, 0, 0)),
            pl.BlockSpec((1, N, 128), lambda b, r: (b, 0, 0)),
        ],
        out_specs=pl.BlockSpec((1, ROW_BLOCK, 128), lambda b, r: (b, r, 0)),
        out_shape=jax.ShapeDtypeStruct((B, N, 128), jnp.float32),
    )(pr, pt, f3)
    return out


# speedup vs baseline: 1.2689x; 1.2689x over previous
import functools

import jax
import jax.numpy as jnp
from jax.experimental import pallas as pl

K_NN = 32
ROW_BLOCK = 512
DROP_BITS = 14


def _mlp_body(pr_ref, w1_ref, b1_ref, w2_ref, b2_ref, w3_ref, b3_ref, f3_ref):
    p = pr_ref[0]
    f = jnp.maximum(jnp.dot(p, w1_ref[...], preferred_element_type=jnp.float32)
                    + b1_ref[...], 0.0)
    f = jnp.maximum(jnp.dot(f, w2_ref[...], preferred_element_type=jnp.float32)
                    + b2_ref[...], 0.0)
    f = jnp.maximum(jnp.dot(f, w3_ref[...], preferred_element_type=jnp.float32)
                    + b3_ref[...], 0.0)
    f3_ref[0] = f.astype(jnp.bfloat16)


def _knn_pool_body(pr_ref, pt_ref, f3_ref, out_ref):
    q = pr_ref[0]
    pt = pt_ref[0]

    d2 = None
    for c in range(3):
        diff = q[:, c:c + 1] - pt[c:c + 1, :]
        sq = diff * diff
        d2 = sq if d2 is None else d2 + sq

    bits = jax.lax.bitcast_convert_type(d2, jnp.int32)
    rb = bits.shape[0]
    n = bits.shape[1]
    ones = jnp.ones((n, 1), jnp.float32)

    hi16 = jnp.right_shift(bits, 16).astype(jnp.int16)

    def body16(i, carry):
        t, m = carry
        cand = t | jnp.left_shift(jnp.int32(1), 14 - i)
        x = (hi16 < cand.astype(jnp.int16)).astype(jnp.int16)
        while x.shape[1] > 128:
            h = x.shape[1] // 2
            x = x[:, :h] + x[:, h:]
        cnt = jnp.sum(x.astype(jnp.int32), axis=1, keepdims=True)
        take = cnt < K_NN
        return jnp.where(take, cand, t), jnp.where(take, cnt, m)

    t16, m16 = jax.lax.fori_loop(
        0, 15, body16,
        (jnp.zeros((rb, 1), jnp.int32), jnp.zeros((rb, 1), jnp.int32)))

    def body(i, carry):
        t, m = carry
        cand = t | jnp.left_shift(jnp.int32(1), 15 - i)
        cnt = jnp.sum((bits < cand).astype(jnp.int32), axis=1, keepdims=True)
        take = cnt < K_NN
        return jnp.where(take, cand, t), jnp.where(take, cnt, m)

    t0 = jnp.left_shift(t16, 16)
    t, m = jax.lax.fori_loop(0, 16 - DROP_BITS, body, (t0, m16))

    ltf = jnp.where(bits < t, 1.0, 0.0)
    lt2f = jnp.where(bits < t + jnp.int32(1 << DROP_BITS), 1.0, 0.0)
    eqf = lt2f - ltf
    e = jnp.sum(eqf, axis=1, keepdims=True)
    tie_w = (float(K_NN) - m.astype(jnp.float32)) / e
    w = (ltf + eqf * tie_w).astype(jnp.bfloat16)

    out_ref[0] = jnp.dot(w, f3_ref[0], preferred_element_type=jnp.float32) \
        * (1.0 / K_NN)


def kernel(points, W1, b1, W2, b2, W3, b3):
    B, N, C = points.shape
    pr = jnp.pad(points, ((0, 0), (0, 0), (0, 8 - C)))
    pt = jnp.swapaxes(pr, 1, 2)
    w1p = jnp.pad(W1, ((0, 8 - C), (0, 0)))

    f3 = pl.pallas_call(
        _mlp_body,
        grid=(B,),
        in_specs=[
            pl.BlockSpec((1, N, 8), lambda b: (b, 0, 0)),
            pl.BlockSpec((8, 64), lambda b: (0, 0)),
            pl.BlockSpec((1, 64), lambda b: (0, 0)),
            pl.BlockSpec((64, 64), lambda b: (0, 0)),
            pl.BlockSpec((1, 64), lambda b: (0, 0)),
            pl.BlockSpec((64, 128), lambda b: (0, 0)),
            pl.BlockSpec((1, 128), lambda b: (0, 0)),
        ],
        out_specs=pl.BlockSpec((1, N, 128), lambda b: (b, 0, 0)),
        out_shape=jax.ShapeDtypeStruct((B, N, 128), jnp.bfloat16),
    )(pr, w1p, b1[None], W2, b2[None], W3, b3[None])

    n_rb = N // ROW_BLOCK
    out = pl.pallas_call(
        _knn_pool_body,
        grid=(B, n_rb),
        in_specs=[
            pl.BlockSpec((1, ROW_BLOCK, 8), lambda b, r: (b, r, 0)),
            pl.BlockSpec((1, 8, N), lambda b, r: (b, 0, 0)),
            pl.BlockSpec((1, N, 128), lambda b, r: (b, 0, 0)),
        ],
        out_specs=pl.BlockSpec((1, ROW_BLOCK, 128), lambda b, r: (b, r, 0)),
        out_shape=jax.ShapeDtypeStruct((B, N, 128), jnp.float32),
    )(pr, pt, f3)
    return out


# speedup vs baseline: 1.3353x; 1.0524x over previous
import functools

import jax
import jax.numpy as jnp
from jax.experimental import pallas as pl

K_NN = 32
ROW_BLOCK = 1024
DROP_BITS = 14


def _mlp_body(pr_ref, w1_ref, b1_ref, w2_ref, b2_ref, w3_ref, b3_ref, f3_ref):
    p = pr_ref[0]
    f = jnp.maximum(jnp.dot(p, w1_ref[...], preferred_element_type=jnp.float32)
                    + b1_ref[...], 0.0)
    f = jnp.maximum(jnp.dot(f, w2_ref[...], preferred_element_type=jnp.float32)
                    + b2_ref[...], 0.0)
    f = jnp.maximum(jnp.dot(f, w3_ref[...], preferred_element_type=jnp.float32)
                    + b3_ref[...], 0.0)
    f3_ref[0] = f.astype(jnp.bfloat16)


def _knn_pool_body(pr_ref, pt_ref, f3_ref, out_ref):
    q = pr_ref[0]
    pt = pt_ref[0]

    d2 = None
    for c in range(3):
        diff = q[:, c:c + 1] - pt[c:c + 1, :]
        sq = diff * diff
        d2 = sq if d2 is None else d2 + sq

    bits = jax.lax.bitcast_convert_type(d2, jnp.int32)
    rb = bits.shape[0]
    n = bits.shape[1]
    ones = jnp.ones((n, 1), jnp.float32)

    hi16 = jnp.right_shift(bits, 16).astype(jnp.int16)

    def body16(i, carry):
        t, m = carry
        cand = t | jnp.left_shift(jnp.int32(1), 14 - i)
        x = (hi16 < cand.astype(jnp.int16)).astype(jnp.int16)
        while x.shape[1] > 128:
            h = x.shape[1] // 2
            x = x[:, :h] + x[:, h:]
        cnt = jnp.sum(x.astype(jnp.int32), axis=1, keepdims=True)
        take = cnt < K_NN
        return jnp.where(take, cand, t), jnp.where(take, cnt, m)

    t16, m16 = jax.lax.fori_loop(
        0, 15, body16,
        (jnp.zeros((rb, 1), jnp.int32), jnp.zeros((rb, 1), jnp.int32)))

    def body(i, carry):
        t, m = carry
        cand = t | jnp.left_shift(jnp.int32(1), 15 - i)
        cnt = jnp.sum((bits < cand).astype(jnp.int32), axis=1, keepdims=True)
        take = cnt < K_NN
        return jnp.where(take, cand, t), jnp.where(take, cnt, m)

    t0 = jnp.left_shift(t16, 16)
    t, m = jax.lax.fori_loop(0, 16 - DROP_BITS, body, (t0, m16))

    ltf = jnp.where(bits < t, 1.0, 0.0)
    lt2f = jnp.where(bits < t + jnp.int32(1 << DROP_BITS), 1.0, 0.0)
    eqf = lt2f - ltf
    e = jnp.sum(eqf, axis=1, keepdims=True)
    tie_w = (float(K_NN) - m.astype(jnp.float32)) / e
    w = (ltf + eqf * tie_w).astype(jnp.bfloat16)

    out_ref[0] = jnp.dot(w, f3_ref[0], preferred_element_type=jnp.float32) \
        * (1.0 / K_NN)


def kernel(points, W1, b1, W2, b2, W3, b3):
    B, N, C = points.shape
    pr = jnp.pad(points, ((0, 0), (0, 0), (0, 8 - C)))
    pt = jnp.swapaxes(pr, 1, 2)
    w1p = jnp.pad(W1, ((0, 8 - C), (0, 0)))

    f3 = pl.pallas_call(
        _mlp_body,
        grid=(B,),
        in_specs=[
            pl.BlockSpec((1, N, 8), lambda b: (b, 0, 0)),
            pl.BlockSpec((8, 64), lambda b: (0, 0)),
            pl.BlockSpec((1, 64), lambda b: (0, 0)),
            pl.BlockSpec((64, 64), lambda b: (0, 0)),
            pl.BlockSpec((1, 64), lambda b: (0, 0)),
            pl.BlockSpec((64, 128), lambda b: (0, 0)),
            pl.BlockSpec((1, 128), lambda b: (0, 0)),
        ],
        out_specs=pl.BlockSpec((1, N, 128), lambda b: (b, 0, 0)),
        out_shape=jax.ShapeDtypeStruct((B, N, 128), jnp.bfloat16),
    )(pr, w1p, b1[None], W2, b2[None], W3, b3[None])

    n_rb = N // ROW_BLOCK
    out = pl.pallas_call(
        _knn_pool_body,
        grid=(B, n_rb),
        in_specs=[
            pl.BlockSpec((1, ROW_BLOCK, 8), lambda b, r: (b, r, 0)),
            pl.BlockSpec((1, 8, N), lambda b, r: (b, 0, 0)),
            pl.BlockSpec((1, N, 128), lambda b, r: (b, 0, 0)),
        ],
        out_specs=pl.BlockSpec((1, ROW_BLOCK, 128), lambda b, r: (b, r, 0)),
        out_shape=jax.ShapeDtypeStruct((B, N, 128), jnp.float32),
    )(pr, pt, f3)
    return out


# speedup vs baseline: 1.3571x; 1.0163x over previous
import functools

import jax
import jax.numpy as jnp
from jax.experimental import pallas as pl

K_NN = 32
ROW_BLOCK = 2048
DROP_BITS = 14


def _mlp_body(pr_ref, w1_ref, b1_ref, w2_ref, b2_ref, w3_ref, b3_ref, f3_ref):
    p = pr_ref[0]
    f = jnp.maximum(jnp.dot(p, w1_ref[...], preferred_element_type=jnp.float32)
                    + b1_ref[...], 0.0)
    f = jnp.maximum(jnp.dot(f, w2_ref[...], preferred_element_type=jnp.float32)
                    + b2_ref[...], 0.0)
    f = jnp.maximum(jnp.dot(f, w3_ref[...], preferred_element_type=jnp.float32)
                    + b3_ref[...], 0.0)
    f3_ref[0] = f.astype(jnp.bfloat16)


def _knn_pool_body(pr_ref, pt_ref, f3_ref, out_ref):
    q = pr_ref[0]
    pt = pt_ref[0]

    d2 = None
    for c in range(3):
        diff = q[:, c:c + 1] - pt[c:c + 1, :]
        sq = diff * diff
        d2 = sq if d2 is None else d2 + sq

    bits = jax.lax.bitcast_convert_type(d2, jnp.int32)
    rb = bits.shape[0]
    n = bits.shape[1]
    ones = jnp.ones((n, 1), jnp.float32)

    hi16 = jnp.right_shift(bits, 16).astype(jnp.int16)

    def body16(i, carry):
        t, m = carry
        cand = t | jnp.left_shift(jnp.int32(1), 14 - i)
        x = (hi16 < cand.astype(jnp.int16)).astype(jnp.int16)
        while x.shape[1] > 128:
            h = x.shape[1] // 2
            x = x[:, :h] + x[:, h:]
        cnt = jnp.sum(x.astype(jnp.int32), axis=1, keepdims=True)
        take = cnt < K_NN
        return jnp.where(take, cand, t), jnp.where(take, cnt, m)

    t16, m16 = jax.lax.fori_loop(
        0, 15, body16,
        (jnp.zeros((rb, 1), jnp.int32), jnp.zeros((rb, 1), jnp.int32)))

    def body(i, carry):
        t, m = carry
        cand = t | jnp.left_shift(jnp.int32(1), 15 - i)
        cnt = jnp.sum((bits < cand).astype(jnp.int32), axis=1, keepdims=True)
        take = cnt < K_NN
        return jnp.where(take, cand, t), jnp.where(take, cnt, m)

    t0 = jnp.left_shift(t16, 16)
    t, m = jax.lax.fori_loop(0, 16 - DROP_BITS, body, (t0, m16))

    ltf = jnp.where(bits < t, 1.0, 0.0)
    lt2f = jnp.where(bits < t + jnp.int32(1 << DROP_BITS), 1.0, 0.0)
    eqf = lt2f - ltf
    e = jnp.sum(eqf, axis=1, keepdims=True)
    tie_w = (float(K_NN) - m.astype(jnp.float32)) / e
    w = (ltf + eqf * tie_w).astype(jnp.bfloat16)

    out_ref[0] = jnp.dot(w, f3_ref[0], preferred_element_type=jnp.float32) \
        * (1.0 / K_NN)


def kernel(points, W1, b1, W2, b2, W3, b3):
    B, N, C = points.shape
    pr = jnp.pad(points, ((0, 0), (0, 0), (0, 8 - C)))
    pt = jnp.swapaxes(pr, 1, 2)
    w1p = jnp.pad(W1, ((0, 8 - C), (0, 0)))

    f3 = pl.pallas_call(
        _mlp_body,
        grid=(B,),
        in_specs=[
            pl.BlockSpec((1, N, 8), lambda b: (b, 0, 0)),
            pl.BlockSpec((8, 64), lambda b: (0, 0)),
            pl.BlockSpec((1, 64), lambda b: (0, 0)),
            pl.BlockSpec((64, 64), lambda b: (0, 0)),
            pl.BlockSpec((1, 64), lambda b: (0, 0)),
            pl.BlockSpec((64, 128), lambda b: (0, 0)),
            pl.BlockSpec((1, 128), lambda b: (0, 0)),
        ],
        out_specs=pl.BlockSpec((1, N, 128), lambda b: (b, 0, 0)),
        out_shape=jax.ShapeDtypeStruct((B, N, 128), jnp.bfloat16),
    )(pr, w1p, b1[None], W2, b2[None], W3, b3[None])

    n_rb = N // ROW_BLOCK
    out = pl.pallas_call(
        _knn_pool_body,
        grid=(B, n_rb),
        in_specs=[
            pl.BlockSpec((1, ROW_BLOCK, 8), lambda b, r: (b, r, 0)),
            pl.BlockSpec((1, 8, N), lambda b, r: (b, 0, 0)),
            pl.BlockSpec((1, N, 128), lambda b, r: (b, 0, 0)),
        ],
        out_specs=pl.BlockSpec((1, ROW_BLOCK, 128), lambda b, r: (b, r, 0)),
        out_shape=jax.ShapeDtypeStruct((B, N, 128), jnp.float32),
    )(pr, pt, f3)
    return out


# speedup vs baseline: 1.3845x; 1.0202x over previous
import jax
import jax.numpy as jnp
from jax.experimental import pallas as pl

K_NN = 32
DROP_BITS = 14


def _fused_body(pr_ref, pt_ref, w1_ref, b1_ref, w2_ref, b2_ref, w3_ref,
                b3_ref, out_ref):
    p = pr_ref[0]
    pt = pt_ref[0]

    f = jnp.maximum(jnp.dot(p, w1_ref[...], preferred_element_type=jnp.float32)
                    + b1_ref[...], 0.0)
    f = jnp.maximum(jnp.dot(f, w2_ref[...], preferred_element_type=jnp.float32)
                    + b2_ref[...], 0.0)
    f = jnp.maximum(jnp.dot(f, w3_ref[...], preferred_element_type=jnp.float32)
                    + b3_ref[...], 0.0)
    f3 = f.astype(jnp.bfloat16)

    d2 = None
    for c in range(3):
        diff = p[:, c:c + 1] - pt[c:c + 1, :]
        sq = diff * diff
        d2 = sq if d2 is None else d2 + sq

    bits = jax.lax.bitcast_convert_type(d2, jnp.int32)
    rb = bits.shape[0]

    hi16 = jnp.right_shift(bits, 16).astype(jnp.int16)

    def body16(i, carry):
        t, m = carry
        cand = t | jnp.left_shift(jnp.int32(1), 14 - i)
        x = (hi16 < cand.astype(jnp.int16)).astype(jnp.int16)
        while x.shape[1] > 128:
            h = x.shape[1] // 2
            x = x[:, :h] + x[:, h:]
        cnt = jnp.sum(x.astype(jnp.int32), axis=1, keepdims=True)
        take = cnt < K_NN
        return jnp.where(take, cand, t), jnp.where(take, cnt, m)

    t16, m16 = jax.lax.fori_loop(
        0, 15, body16,
        (jnp.zeros((rb, 1), jnp.int32), jnp.zeros((rb, 1), jnp.int32)))

    def body(i, carry):
        t, m = carry
        cand = t | jnp.left_shift(jnp.int32(1), 15 - i)
        cnt = jnp.sum((bits < cand).astype(jnp.int32), axis=1, keepdims=True)
        take = cnt < K_NN
        return jnp.where(take, cand, t), jnp.where(take, cnt, m)

    t0 = jnp.left_shift(t16, 16)
    t, m = jax.lax.fori_loop(0, 16 - DROP_BITS, body, (t0, m16))

    ltf = jnp.where(bits < t, 1.0, 0.0)
    lt2f = jnp.where(bits < t + jnp.int32(1 << DROP_BITS), 1.0, 0.0)
    eqf = lt2f - ltf
    e = jnp.sum(eqf, axis=1, keepdims=True)
    tie_w = (float(K_NN) - m.astype(jnp.float32)) / e
    w = (ltf + eqf * tie_w).astype(jnp.bfloat16)

    out_ref[0] = jnp.dot(w, f3, preferred_element_type=jnp.float32) \
        * (1.0 / K_NN)


def kernel(points, W1, b1, W2, b2, W3, b3):
    B, N, C = points.shape
    pr = jnp.pad(points, ((0, 0), (0, 0), (0, 8 - C)))
    pt = jnp.swapaxes(pr, 1, 2)
    w1p = jnp.pad(W1, ((0, 8 - C), (0, 0)))

    out = pl.pallas_call(
        _fused_body,
        grid=(B,),
        in_specs=[
            pl.BlockSpec((1, N, 8), lambda b: (b, 0, 0)),
            pl.BlockSpec((1, 8, N), lambda b: (b, 0, 0)),
            pl.BlockSpec((8, 64), lambda b: (0, 0)),
            pl.BlockSpec((1, 64), lambda b: (0, 0)),
            pl.BlockSpec((64, 64), lambda b: (0, 0)),
            pl.BlockSpec((1, 64), lambda b: (0, 0)),
            pl.BlockSpec((64, 128), lambda b: (0, 0)),
            pl.BlockSpec((1, 128), lambda b: (0, 0)),
        ],
        out_specs=pl.BlockSpec((1, N, 128), lambda b: (b, 0, 0)),
        out_shape=jax.ShapeDtypeStruct((B, N, 128), jnp.float32),
    )(pr, pt, w1p, b1[None], W2, b2[None], W3, b3[None])
    return out


# speedup vs baseline: 1.4766x; 1.0665x over previous
import jax
import jax.numpy as jnp
from jax.experimental import pallas as pl

K_NN = 32
DROP_BITS = 15


def _fused_body(pr_ref, pt_ref, w1_ref, b1_ref, w2_ref, b2_ref, w3_ref,
                b3_ref, out_ref):
    p = pr_ref[0]
    pt = pt_ref[0]

    f = jnp.maximum(jnp.dot(p, w1_ref[...], preferred_element_type=jnp.float32)
                    + b1_ref[...], 0.0)
    f = jnp.maximum(jnp.dot(f, w2_ref[...], preferred_element_type=jnp.float32)
                    + b2_ref[...], 0.0)
    f = jnp.maximum(jnp.dot(f, w3_ref[...], preferred_element_type=jnp.float32)
                    + b3_ref[...], 0.0)
    f3 = f.astype(jnp.bfloat16)

    d2 = None
    for c in range(3):
        diff = p[:, c:c + 1] - pt[c:c + 1, :]
        sq = diff * diff
        d2 = sq if d2 is None else d2 + sq

    bits = jax.lax.bitcast_convert_type(d2, jnp.int32)
    rb = bits.shape[0]

    hi16 = jnp.right_shift(bits, 16).astype(jnp.int16)

    def body16(i, carry):
        t, m = carry
        cand = t | jnp.left_shift(jnp.int32(1), 14 - i)
        x = (hi16 < cand.astype(jnp.int16)).astype(jnp.int16)
        while x.shape[1] > 128:
            h = x.shape[1] // 2
            x = x[:, :h] + x[:, h:]
        cnt = jnp.sum(x.astype(jnp.int32), axis=1, keepdims=True)
        take = cnt < K_NN
        return jnp.where(take, cand, t), jnp.where(take, cnt, m)

    t16, m16 = jax.lax.fori_loop(
        0, 15, body16,
        (jnp.zeros((rb, 1), jnp.int32), jnp.zeros((rb, 1), jnp.int32)))

    def body(i, carry):
        t, m = carry
        cand = t | jnp.left_shift(jnp.int32(1), 15 - i)
        cnt = jnp.sum((bits < cand).astype(jnp.int32), axis=1, keepdims=True)
        take = cnt < K_NN
        return jnp.where(take, cand, t), jnp.where(take, cnt, m)

    t0 = jnp.left_shift(t16, 16)
    t, m = jax.lax.fori_loop(0, 16 - DROP_BITS, body, (t0, m16))

    ltf = jnp.where(bits < t, 1.0, 0.0)
    lt2f = jnp.where(bits < t + jnp.int32(1 << DROP_BITS), 1.0, 0.0)
    eqf = lt2f - ltf
    e = jnp.sum(eqf, axis=1, keepdims=True)
    tie_w = (float(K_NN) - m.astype(jnp.float32)) / e
    w = (ltf + eqf * tie_w).astype(jnp.bfloat16)

    out_ref[0] = jnp.dot(w, f3, preferred_element_type=jnp.float32) \
        * (1.0 / K_NN)


def kernel(points, W1, b1, W2, b2, W3, b3):
    B, N, C = points.shape
    pr = jnp.pad(points, ((0, 0), (0, 0), (0, 8 - C)))
    pt = jnp.swapaxes(pr, 1, 2)
    w1p = jnp.pad(W1, ((0, 8 - C), (0, 0)))

    out = pl.pallas_call(
        _fused_body,
        grid=(B,),
        in_specs=[
            pl.BlockSpec((1, N, 8), lambda b: (b, 0, 0)),
            pl.BlockSpec((1, 8, N), lambda b: (b, 0, 0)),
            pl.BlockSpec((8, 64), lambda b: (0, 0)),
            pl.BlockSpec((1, 64), lambda b: (0, 0)),
            pl.BlockSpec((64, 64), lambda b: (0, 0)),
            pl.BlockSpec((1, 64), lambda b: (0, 0)),
            pl.BlockSpec((64, 128), lambda b: (0, 0)),
            pl.BlockSpec((1, 128), lambda b: (0, 0)),
        ],
        out_specs=pl.BlockSpec((1, N, 128), lambda b: (b, 0, 0)),
        out_shape=jax.ShapeDtypeStruct((B, N, 128), jnp.float32),
    )(pr, pt, w1p, b1[None], W2, b2[None], W3, b3[None])
    return out
